# baseline (speedup 1.0000x reference)
import functools

import jax
import jax.numpy as jnp
from jax import lax
from jax.experimental import pallas as pl
from jax.experimental.pallas import tpu as pltpu
from jax.experimental.pallas import tpu_sc as plsc


def _fuse_body(a_ref, w_ref, b_ref, o_ref):
    o_ref[...] = w_ref[...] + jnp.dot(
        a_ref[...], b_ref[...], preferred_element_type=jnp.float32
    )


def _fuse_table(weight, A, B, rows_per_block=1000):
    vocab, dim = weight.shape
    rank = A.shape[1]
    grid = pl.cdiv(vocab, rows_per_block)
    return pl.pallas_call(
        _fuse_body,
        grid=(grid,),
        in_specs=[
            pl.BlockSpec((rows_per_block, rank), lambda i: (i, 0)),
            pl.BlockSpec((rows_per_block, dim), lambda i: (i, 0)),
            pl.BlockSpec((rank, dim), lambda i: (0, 0)),
        ],
        out_specs=pl.BlockSpec((rows_per_block, dim), lambda i: (i, 0)),
        out_shape=jax.ShapeDtypeStruct((vocab, dim), jnp.float32),
    )(A, weight, B)


def _sc_gather(table, idx3d, m=4):
    _, batch, hpad = idx3d.shape
    dim = table.shape[1]
    info = plsc.get_sparse_core_info()
    nc, ns = info.num_cores, info.num_subcores
    nw = nc * ns
    b_per_w = batch // nw
    n_c = b_per_w // m
    mesh = plsc.VectorSubcoreMesh(core_axis_name="c", subcore_axis_name="s")
    idx4d = idx3d.reshape(2, nw, 1, b_per_w * hpad)
    table3 = table.reshape(1, table.shape[0], dim)

    @functools.partial(
        pl.kernel,
        mesh=mesh,
        compiler_params=pltpu.CompilerParams(use_tc_tiling_on_sc=False),
        out_type=jax.ShapeDtypeStruct(
            (2, nw, n_c, 1, m * hpad, dim), jnp.float32
        ),
        scratch_types=[
            pltpu.VMEM((2, 1, b_per_w * hpad), jnp.int32),
            pltpu.VMEM((2, 1, m * hpad, dim), jnp.float32),
            pltpu.SemaphoreType.DMA,
            pltpu.SemaphoreType.DMA,
        ],
    )
    def gather_kernel(table_hbm, idx_hbm, out_hbm, idx_v, rows_v, gsem, ssem):
        wid = lax.axis_index("s") * nc + lax.axis_index("c")
        for h in range(2):
            pltpu.sync_copy(idx_hbm.at[h, wid], idx_v.at[h])

        def g_copy(ring, c):
            h = c // n_c
            cc = c % n_c
            return pltpu.make_async_copy(
                table_hbm.at[idx_v.at[h, :, pl.ds(cc * m * hpad, m * hpad)]],
                rows_v.at[ring],
                gsem,
            )


        def s_copy(ring, c):
            h = c // n_c
            cc = c % n_c
            return pltpu.make_async_copy(
                rows_v.at[ring],
                out_hbm.at[h, wid, cc],
                ssem,
            )

        g_copy(0, 0).start()

        def body(t, carry):
            ca = 2 * t
            cb = ca + 1
            cc = ca + 2
            g_copy(0, ca).wait()
            s_copy(0, ca).start()

            @pl.when(t > 0)
            def _():
                s_copy(1, cb - 2).wait()

            g_copy(1, cb).start()
            g_copy(1, cb).wait()
            s_copy(1, cb).start()
            s_copy(0, ca).wait()

            @pl.when(t + 1 < n_c)
            def _():
                g_copy(0, cc).start()

            return carry

        lax.fori_loop(0, n_c, body, 0)
        s_copy(1, 2 * n_c - 1).wait()

    return gather_kernel(table3, idx4d)


def _xpose_body(x_ref, o_ref):
    x = x_ref[...]
    hp, dim, bb = o_ref.shape
    for p in range(hp // 2):
        xt = x[:, p, :].T
        o_ref[2 * p] = xt[0:dim]
        o_ref[2 * p + 1] = xt[dim : 2 * dim]


def _tc_transpose(packed, batch, hist, dim, bb=128):
    hpad = packed.size // (2 * batch * dim)
    xx = packed.reshape(2 * batch, hpad // 2, 2 * dim)
    nb = batch // bb
    grid = (2 * nb,)
    out = pl.pallas_call(
        _xpose_body,
        grid=grid,
        in_specs=[
            pl.BlockSpec((bb, hpad // 2, 2 * dim), lambda g: (g, 0, 0))
        ],
        out_specs=pl.BlockSpec(
            (hist // 2, dim, bb), lambda g: (g // nb, 0, g % nb)
        ),
        out_shape=jax.ShapeDtypeStruct((hist, dim, batch), jnp.float32),
    )(xx)
    return jnp.transpose(out, (2, 0, 1))


def kernel(indices, weight, A, B):
    batch, hist = indices.shape
    dim = weight.shape[1]
    hp = hist // 2
    hpad = hp + (-hp) % 16
    idx3d = jnp.pad(
        indices.astype(jnp.int32).reshape(batch, 2, hp).transpose(1, 0, 2),
        ((0, 0), (0, 0), (0, hpad - hp)),
    )
    fused = _fuse_table(weight, A, B)
    packed = _sc_gather(fused, idx3d)
    return _tc_transpose(packed, batch, hist, dim)


# speedup vs baseline: 1.0005x; 1.0005x over previous
import functools

import jax
import jax.numpy as jnp
from jax import lax
from jax.experimental import pallas as pl
from jax.experimental.pallas import tpu as pltpu
from jax.experimental.pallas import tpu_sc as plsc


def _fuse_body(a_ref, w_ref, b_ref, o_ref):
    o_ref[...] = w_ref[...] + jnp.dot(
        a_ref[...], b_ref[...], preferred_element_type=jnp.float32
    )


def _fuse_table(weight, A, B, rows_per_block=1000):
    vocab, dim = weight.shape
    rank = A.shape[1]
    grid = pl.cdiv(vocab, rows_per_block)
    return pl.pallas_call(
        _fuse_body,
        grid=(grid,),
        in_specs=[
            pl.BlockSpec((rows_per_block, rank), lambda i: (i, 0)),
            pl.BlockSpec((rows_per_block, dim), lambda i: (i, 0)),
            pl.BlockSpec((rank, dim), lambda i: (0, 0)),
        ],
        out_specs=pl.BlockSpec((rows_per_block, dim), lambda i: (i, 0)),
        out_shape=jax.ShapeDtypeStruct((vocab, dim), jnp.float32),
    )(A, weight, B)


def _sc_gather(table, idx3d, m=4):
    _, batch, hpad = idx3d.shape
    dim = table.shape[1]
    info = plsc.get_sparse_core_info()
    nc, ns = info.num_cores, info.num_subcores
    nw = nc * ns
    b_per_w = batch // nw
    n_c = b_per_w // m
    mesh = plsc.VectorSubcoreMesh(core_axis_name="c", subcore_axis_name="s")
    idx4d = idx3d.reshape(2, nw, b_per_w * hpad)

    @functools.partial(
        pl.kernel,
        mesh=mesh,
        compiler_params=pltpu.CompilerParams(use_tc_tiling_on_sc=False),
        out_type=jax.ShapeDtypeStruct(
            (2, nw, n_c, m * hpad, dim), jnp.float32
        ),
        scratch_types=[
            pltpu.VMEM((2, b_per_w * hpad), jnp.int32),
            pltpu.VMEM((2, m * hpad, dim), jnp.float32),
            pltpu.SemaphoreType.DMA,
            pltpu.SemaphoreType.DMA,
        ],
    )
    def gather_kernel(table_hbm, idx_hbm, out_hbm, idx_v, rows_v, gsem, ssem):
        wid = lax.axis_index("s") * nc + lax.axis_index("c")
        for h in range(2):
            pltpu.sync_copy(idx_hbm.at[h, wid], idx_v.at[h])

        def g_copy(ring, c):
            h = c // n_c
            cc = c % n_c
            return pltpu.make_async_copy(
                table_hbm.at[idx_v.at[h, pl.ds(cc * m * hpad, m * hpad)]],
                rows_v.at[ring],
                gsem,
            )


        def s_copy(ring, c):
            h = c // n_c
            cc = c % n_c
            return pltpu.make_async_copy(
                rows_v.at[ring],
                out_hbm.at[h, wid, cc],
                ssem,
            )

        g_copy(0, 0).start()

        def body(t, carry):
            ca = 2 * t
            cb = ca + 1
            cc = ca + 2
            g_copy(0, ca).wait()
            s_copy(0, ca).start()

            @pl.when(t > 0)
            def _():
                s_copy(1, cb - 2).wait()

            g_copy(1, cb).start()
            g_copy(1, cb).wait()
            s_copy(1, cb).start()
            s_copy(0, ca).wait()

            @pl.when(t + 1 < n_c)
            def _():
                g_copy(0, cc).start()

            return carry

        lax.fori_loop(0, n_c, body, 0)
        s_copy(1, 2 * n_c - 1).wait()

    return gather_kernel(table, idx4d)


def _xpose_body(x_ref, o_ref):
    x = x_ref[...]
    hp, dim, bb = o_ref.shape
    for p in range(hp // 2):
        xt = x[:, p, :].T
        o_ref[2 * p] = xt[0:dim]
        o_ref[2 * p + 1] = xt[dim : 2 * dim]


def _tc_transpose(packed, batch, hist, dim, bb=128):
    hpad = packed.size // (2 * batch * dim)
    xx = packed.reshape(2 * batch, hpad // 2, 2 * dim)
    nb = batch // bb
    grid = (2 * nb,)
    out = pl.pallas_call(
        _xpose_body,
        grid=grid,
        in_specs=[
            pl.BlockSpec((bb, hpad // 2, 2 * dim), lambda g: (g, 0, 0))
        ],
        out_specs=pl.BlockSpec(
            (hist // 2, dim, bb), lambda g: (g // nb, 0, g % nb)
        ),
        out_shape=jax.ShapeDtypeStruct((hist, dim, batch), jnp.float32),
    )(xx)
    return jnp.transpose(out, (2, 0, 1))


def kernel(indices, weight, A, B):
    batch, hist = indices.shape
    dim = weight.shape[1]
    hp = hist // 2
    hpad = hp + (-hp) % 16
    idx3d = jnp.pad(
        indices.astype(jnp.int32).reshape(batch, 2, hp).transpose(1, 0, 2),
        ((0, 0), (0, 0), (0, hpad - hp)),
    )
    fused = _fuse_table(weight, A, B)
    packed = _sc_gather(fused, idx3d)
    return _tc_transpose(packed, batch, hist, dim)


# speedup vs baseline: 1.0020x; 1.0015x over previous
import functools

import jax
import jax.numpy as jnp
from jax import lax
from jax.experimental import pallas as pl
from jax.experimental.pallas import tpu as pltpu
from jax.experimental.pallas import tpu_sc as plsc


def _fuse_body(a_ref, w_ref, b_ref, o_ref):
    o_ref[...] = w_ref[...] + jnp.dot(
        a_ref[...], b_ref[...], preferred_element_type=jnp.float32
    )


def _fuse_table(weight, A, B, rows_per_block=1000):
    vocab, dim = weight.shape
    rank = A.shape[1]
    grid = pl.cdiv(vocab, rows_per_block)
    return pl.pallas_call(
        _fuse_body,
        grid=(grid,),
        in_specs=[
            pl.BlockSpec((rows_per_block, rank), lambda i: (i, 0)),
            pl.BlockSpec((rows_per_block, dim), lambda i: (i, 0)),
            pl.BlockSpec((rank, dim), lambda i: (0, 0)),
        ],
        out_specs=pl.BlockSpec((rows_per_block, dim), lambda i: (i, 0)),
        out_shape=jax.ShapeDtypeStruct((vocab, dim), jnp.float32),
    )(A, weight, B)


def _sc_gather(table, idx3d, m=8):
    _, batch, hpad = idx3d.shape
    dim = table.shape[1]
    info = plsc.get_sparse_core_info()
    nc, ns = info.num_cores, info.num_subcores
    nw = nc * ns
    b_per_w = batch // nw
    n_c = b_per_w // m
    mesh = plsc.VectorSubcoreMesh(core_axis_name="c", subcore_axis_name="s")
    idx4d = idx3d.reshape(2, nw, b_per_w, hpad)

    @functools.partial(
        pl.kernel,
        mesh=mesh,
        compiler_params=pltpu.CompilerParams(use_tc_tiling_on_sc=False),
        out_type=jax.ShapeDtypeStruct(
            (2, nw, n_c, m, hpad, dim), jnp.float32
        ),
        scratch_types=[
            pltpu.VMEM((2, b_per_w, hpad), jnp.int32),
            pltpu.VMEM((m, hpad, dim), jnp.float32),
            pltpu.SemaphoreType.DMA,
            pltpu.SemaphoreType.DMA,
        ],
    )
    def gather_kernel(table_hbm, idx_hbm, out_hbm, idx_v, rows_v, gsem, ssem):
        wid = lax.axis_index("s") * nc + lax.axis_index("c")
        for h in range(2):
            pltpu.sync_copy(idx_hbm.at[h, wid], idx_v.at[h])

        def body(c, carry):
            h = c // n_c
            cc = c % n_c
            for q in range(m):
                pltpu.async_copy(
                    table_hbm.at[idx_v.at[h, cc * m + q]],
                    rows_v.at[q],
                    gsem,
                )
            for q in range(m):
                pltpu.make_async_copy(
                    table_hbm.at[idx_v.at[h, cc * m + q]],
                    rows_v.at[q],
                    gsem,
                ).wait()
            pltpu.async_copy(rows_v, out_hbm.at[h, wid, cc], ssem)
            pltpu.make_async_copy(
                rows_v, out_hbm.at[h, wid, cc], ssem
            ).wait()
            return carry

        lax.fori_loop(0, 2 * n_c, body, 0)

    return gather_kernel(table, idx4d)


def _xpose_body(x_ref, o_ref):
    x = x_ref[...]
    hp, dim, bb = o_ref.shape
    for p in range(hp // 2):
        xt = x[:, p, :].T
        o_ref[2 * p] = xt[0:dim]
        o_ref[2 * p + 1] = xt[dim : 2 * dim]


def _tc_transpose(packed, batch, hist, dim, bb=128):
    hpad = packed.size // (2 * batch * dim)
    xx = packed.reshape(2 * batch, hpad // 2, 2 * dim)
    nb = batch // bb
    grid = (2 * nb,)
    out = pl.pallas_call(
        _xpose_body,
        grid=grid,
        in_specs=[
            pl.BlockSpec((bb, hpad // 2, 2 * dim), lambda g: (g, 0, 0))
        ],
        out_specs=pl.BlockSpec(
            (hist // 2, dim, bb), lambda g: (g // nb, 0, g % nb)
        ),
        out_shape=jax.ShapeDtypeStruct((hist, dim, batch), jnp.float32),
    )(xx)
    return jnp.transpose(out, (2, 0, 1))


def kernel(indices, weight, A, B):
    batch, hist = indices.shape
    dim = weight.shape[1]
    hp = hist // 2
    hpad = hp + (-hp) % 16
    idx3d = jnp.pad(
        indices.astype(jnp.int32).reshape(batch, 2, hp).transpose(1, 0, 2),
        ((0, 0), (0, 0), (0, hpad - hp)),
    )
    fused = _fuse_table(weight, A, B)
    packed = _sc_gather(fused, idx3d)
    return _tc_transpose(packed, batch, hist, dim)


# speedup vs baseline: 4.9483x; 4.9387x over previous
import functools

import jax
import jax.numpy as jnp
from jax import lax
from jax.experimental import pallas as pl
from jax.experimental.pallas import tpu as pltpu
from jax.experimental.pallas import tpu_sc as plsc


def _fuse_body(a_ref, w_ref, b_ref, o_ref):
    o_ref[...] = w_ref[...] + jnp.dot(
        a_ref[...], b_ref[...], preferred_element_type=jnp.float32
    )


def _fuse_table(weight, A, B, rows_per_block=1000):
    vocab, dim = weight.shape
    rank = A.shape[1]
    grid = pl.cdiv(vocab, rows_per_block)
    return pl.pallas_call(
        _fuse_body,
        grid=(grid,),
        in_specs=[
            pl.BlockSpec((rows_per_block, rank), lambda i: (i, 0)),
            pl.BlockSpec((rows_per_block, dim), lambda i: (i, 0)),
            pl.BlockSpec((rank, dim), lambda i: (0, 0)),
        ],
        out_specs=pl.BlockSpec((rows_per_block, dim), lambda i: (i, 0)),
        out_shape=jax.ShapeDtypeStruct((vocab, dim), jnp.float32),
    )(A, weight, B)


def _sc_gather(table, idx3d, m=8):
    _, batch, hpad = idx3d.shape
    dim = table.shape[1]
    info = plsc.get_sparse_core_info()
    nc, ns = info.num_cores, info.num_subcores
    nw = nc * ns
    b_per_w = batch // nw
    n_c = b_per_w // m
    mesh = plsc.VectorSubcoreMesh(core_axis_name="c", subcore_axis_name="s")
    idx4d = idx3d.reshape(2, nw, b_per_w, hpad)

    @functools.partial(
        pl.kernel,
        mesh=mesh,
        compiler_params=pltpu.CompilerParams(use_tc_tiling_on_sc=False),
        out_type=jax.ShapeDtypeStruct(
            (2, nw, n_c, m, hpad, dim), jnp.float32
        ),
        scratch_types=[
            pltpu.VMEM((2, b_per_w, hpad), jnp.int32),
            pltpu.VMEM((m, hpad, dim), jnp.float32),
            pltpu.SemaphoreType.DMA,
            pltpu.SemaphoreType.DMA,
        ],
    )
    def gather_kernel(table_hbm, idx_hbm, out_hbm, idx_v, rows_v, gsem, ssem):
        wid = lax.axis_index("s") * nc + lax.axis_index("c")
        for h in range(2):
            pltpu.sync_copy(idx_hbm.at[h, wid], idx_v.at[h])

        def body(c, carry):
            h = c // n_c
            cc = c % n_c
            for q in range(m):
                pltpu.async_copy(
                    table_hbm.at[idx_v.at[h, cc * m + q]],
                    rows_v.at[q],
                    gsem,
                )
            for q in range(m):
                pltpu.make_async_copy(
                    table_hbm.at[idx_v.at[h, cc * m + q]],
                    rows_v.at[q],
                    gsem,
                ).wait()
            pltpu.async_copy(rows_v, out_hbm.at[h, wid, cc], ssem)
            pltpu.make_async_copy(
                rows_v, out_hbm.at[h, wid, cc], ssem
            ).wait()
            return carry

        lax.fori_loop(0, 2 * n_c, body, 0)

    return gather_kernel(table, idx4d)


def _xpose_body(x_ref, o_ref):
    x = x_ref[...]
    hp, dim, bb = o_ref.shape
    for p in range(hp // 2):
        xt = x[:, p, :].T
        o_ref[2 * p] = xt[0:dim]
        o_ref[2 * p + 1] = xt[dim : 2 * dim]


def _tc_transpose(packed, batch, hist, dim, bb=128):
    hpad = packed.size // (2 * batch * dim)
    xx = packed.reshape(2 * batch, hpad // 2, 2 * dim)
    nb = batch // bb
    grid = (2 * nb,)
    out = pl.pallas_call(
        _xpose_body,
        grid=grid,
        in_specs=[
            pl.BlockSpec((bb, hpad // 2, 2 * dim), lambda g: (g, 0, 0))
        ],
        out_specs=pl.BlockSpec(
            (hist // 2, dim, bb), lambda g: (g // nb, 0, g % nb)
        ),
        out_shape=jax.ShapeDtypeStruct((hist, dim, batch), jnp.float32),
    )(xx)
    return jnp.transpose(out, (2, 0, 1))


def kernel(indices, weight, A, B):
    batch, hist = indices.shape
    dim = weight.shape[1]
    hp = hist // 2
    hpad = hp + (-hp) % 16
    idxh = indices.astype(jnp.int32).reshape(batch, 2, hp).transpose(1, 0, 2)
    idx3d = jnp.concatenate([idxh, idxh[:, :, : hpad - hp]], axis=2)
    fused = _fuse_table(weight, A, B)
    packed = _sc_gather(fused, idx3d)
    return _tc_transpose(packed, batch, hist, dim)


# speedup vs baseline: 5.4155x; 1.0944x over previous
import functools

import jax
import jax.numpy as jnp
from jax import lax
from jax.experimental import pallas as pl
from jax.experimental.pallas import tpu as pltpu
from jax.experimental.pallas import tpu_sc as plsc


def _fuse_body(a_ref, w_ref, b_ref, o_ref):
    o_ref[...] = w_ref[...] + jnp.dot(
        a_ref[...], b_ref[...], preferred_element_type=jnp.float32
    )


def _fuse_table(weight, A, B, rows_per_block=2000):
    vocab, dim = weight.shape
    rank = A.shape[1]
    grid = pl.cdiv(vocab, rows_per_block)
    return pl.pallas_call(
        _fuse_body,
        grid=(grid,),
        in_specs=[
            pl.BlockSpec((rows_per_block, rank), lambda i: (i, 0)),
            pl.BlockSpec((rows_per_block, dim), lambda i: (i, 0)),
            pl.BlockSpec((rank, dim), lambda i: (0, 0)),
        ],
        out_specs=pl.BlockSpec((rows_per_block, dim), lambda i: (i, 0)),
        out_shape=jax.ShapeDtypeStruct((vocab, dim), jnp.float32),
    )(A, weight, B)


def _sc_gather(table, idx3d, m=8):
    _, batch, hpad = idx3d.shape
    dim = table.shape[1]
    info = plsc.get_sparse_core_info()
    nc, ns = info.num_cores, info.num_subcores
    nw = nc * ns
    b_per_w = batch // nw
    n_c = b_per_w // m
    mesh = plsc.VectorSubcoreMesh(core_axis_name="c", subcore_axis_name="s")
    idx4d = idx3d.reshape(2, nw, b_per_w, hpad)

    @functools.partial(
        pl.kernel,
        mesh=mesh,
        compiler_params=pltpu.CompilerParams(use_tc_tiling_on_sc=False),
        out_type=jax.ShapeDtypeStruct(
            (2, nw, n_c, m, hpad, dim), jnp.float32
        ),
        scratch_types=[
            pltpu.VMEM((2, b_per_w, hpad), jnp.int32),
            pltpu.VMEM((m, hpad, dim), jnp.float32),
            pltpu.SemaphoreType.DMA,
            pltpu.SemaphoreType.DMA,
        ],
    )
    def gather_kernel(table_hbm, idx_hbm, out_hbm, idx_v, rows_v, gsem, ssem):
        wid = lax.axis_index("s") * nc + lax.axis_index("c")
        for h in range(2):
            pltpu.sync_copy(idx_hbm.at[h, wid], idx_v.at[h])

        def body(c, carry):
            h = c // n_c
            cc = c % n_c
            for q in range(m):
                pltpu.async_copy(
                    table_hbm.at[idx_v.at[h, cc * m + q]],
                    rows_v.at[q],
                    gsem,
                )
            for q in range(m):
                pltpu.make_async_copy(
                    table_hbm.at[idx_v.at[h, cc * m + q]],
                    rows_v.at[q],
                    gsem,
                ).wait()
            pltpu.async_copy(rows_v, out_hbm.at[h, wid, cc], ssem)
            pltpu.make_async_copy(
                rows_v, out_hbm.at[h, wid, cc], ssem
            ).wait()
            return carry

        lax.fori_loop(0, 2 * n_c, body, 0)

    return gather_kernel(table, idx4d)


def _xpose_body(x_ref, o_ref):
    x = x_ref[...]
    hp, dim, bb = o_ref.shape
    for p in range(hp // 2):
        xt = x[:, p, :].T
        o_ref[2 * p] = xt[0:dim]
        o_ref[2 * p + 1] = xt[dim : 2 * dim]


def _tc_transpose(packed, batch, hist, dim, bb=256):
    hpad = packed.size // (2 * batch * dim)
    xx = packed.reshape(2 * batch, hpad // 2, 2 * dim)
    nb = batch // bb
    grid = (2 * nb,)
    out = pl.pallas_call(
        _xpose_body,
        grid=grid,
        in_specs=[
            pl.BlockSpec((bb, hpad // 2, 2 * dim), lambda g: (g, 0, 0))
        ],
        out_specs=pl.BlockSpec(
            (hist // 2, dim, bb), lambda g: (g // nb, 0, g % nb)
        ),
        out_shape=jax.ShapeDtypeStruct((hist, dim, batch), jnp.float32),
    )(xx)
    return jnp.transpose(out, (2, 0, 1))


def kernel(indices, weight, A, B):
    batch, hist = indices.shape
    dim = weight.shape[1]
    hp = hist // 2
    hpad = hp + (-hp) % 16
    idxh = indices.astype(jnp.int32).reshape(batch, 2, hp).transpose(1, 0, 2)
    idx3d = jnp.concatenate([idxh, idxh[:, :, : hpad - hp]], axis=2)
    fused = _fuse_table(weight, A, B)
    packed = _sc_gather(fused, idx3d)
    return _tc_transpose(packed, batch, hist, dim)


# speedup vs baseline: 5.5601x; 1.0267x over previous
import functools

import jax
import jax.numpy as jnp
from jax import lax
from jax.experimental import pallas as pl
from jax.experimental.pallas import tpu as pltpu
from jax.experimental.pallas import tpu_sc as plsc


def _fuse_body(a_ref, w_ref, b_ref, o_ref):
    o_ref[...] = w_ref[...] + jnp.dot(
        a_ref[...], b_ref[...], preferred_element_type=jnp.float32
    )


def _fuse_table(weight, A, B, rows_per_block=2000):
    vocab, dim = weight.shape
    rank = A.shape[1]
    grid = pl.cdiv(vocab, rows_per_block)
    return pl.pallas_call(
        _fuse_body,
        grid=(grid,),
        in_specs=[
            pl.BlockSpec((rows_per_block, rank), lambda i: (i, 0)),
            pl.BlockSpec((rows_per_block, dim), lambda i: (i, 0)),
            pl.BlockSpec((rank, dim), lambda i: (0, 0)),
        ],
        out_specs=pl.BlockSpec((rows_per_block, dim), lambda i: (i, 0)),
        out_shape=jax.ShapeDtypeStruct((vocab, dim), jnp.float32),
    )(A, weight, B)


def _sc_gather(table, idx3d, m=4):
    _, batch, hpad = idx3d.shape
    dim = table.shape[1]
    info = plsc.get_sparse_core_info()
    nc, ns = info.num_cores, info.num_subcores
    nw = nc * ns
    b_per_w = batch // nw
    n_c = b_per_w // m
    mesh = plsc.VectorSubcoreMesh(core_axis_name="c", subcore_axis_name="s")
    idx4d = idx3d.reshape(2, nw, b_per_w, hpad)

    @functools.partial(
        pl.kernel,
        mesh=mesh,
        compiler_params=pltpu.CompilerParams(use_tc_tiling_on_sc=False),
        out_type=jax.ShapeDtypeStruct(
            (2, nw, n_c, m, hpad, dim), jnp.float32
        ),
        scratch_types=[
            pltpu.VMEM((2, b_per_w, hpad), jnp.int32),
            pltpu.VMEM((2, m, hpad, dim), jnp.float32),
            pltpu.SemaphoreType.DMA,
            pltpu.SemaphoreType.DMA,
        ],
    )
    def gather_kernel(table_hbm, idx_hbm, out_hbm, idx_v, rows_v, gsem, ssem):
        wid = lax.axis_index("s") * nc + lax.axis_index("c")
        for h in range(2):
            pltpu.sync_copy(idx_hbm.at[h, wid], idx_v.at[h])

        n_t = 2 * n_c

        def fire_g(ring, c):
            h = c // n_c
            cc = c % n_c
            for q in range(m):
                pltpu.async_copy(
                    table_hbm.at[idx_v.at[h, cc * m + q]],
                    rows_v.at[ring, q],
                    gsem,
                )

        def wait_g(ring, c):
            h = c // n_c
            cc = c % n_c
            for q in range(m):
                pltpu.make_async_copy(
                    table_hbm.at[idx_v.at[h, cc * m + q]],
                    rows_v.at[ring, q],
                    gsem,
                ).wait()

        def s_copy(ring, c):
            h = c // n_c
            cc = c % n_c
            return pltpu.make_async_copy(
                rows_v.at[ring], out_hbm.at[h, wid, cc], ssem
            )

        fire_g(0, 0)

        def body(t, carry):
            ca = 2 * t
            cb = ca + 1
            wait_g(0, ca)
            fire_g(1, cb)
            s_copy(0, ca).start()
            wait_g(1, cb)
            s_copy(1, cb).start()
            s_copy(0, ca).wait()

            @pl.when(t + 1 < n_t // 2)
            def _():
                fire_g(0, ca + 2)

            s_copy(1, cb).wait()
            return carry

        lax.fori_loop(0, n_t // 2, body, 0)

    return gather_kernel(table, idx4d)


def _xpose_body(x_ref, o_ref):
    x = x_ref[...]
    hp, dim, bb = o_ref.shape
    for p in range(hp // 2):
        xt = x[:, p, :].T
        o_ref[2 * p] = xt[0:dim]
        o_ref[2 * p + 1] = xt[dim : 2 * dim]


def _tc_transpose(packed, batch, hist, dim, bb=256):
    hpad = packed.size // (2 * batch * dim)
    xx = packed.reshape(2 * batch, hpad // 2, 2 * dim)
    nb = batch // bb
    grid = (2 * nb,)
    out = pl.pallas_call(
        _xpose_body,
        grid=grid,
        in_specs=[
            pl.BlockSpec((bb, hpad // 2, 2 * dim), lambda g: (g, 0, 0))
        ],
        out_specs=pl.BlockSpec(
            (hist // 2, dim, bb), lambda g: (g // nb, 0, g % nb)
        ),
        out_shape=jax.ShapeDtypeStruct((hist, dim, batch), jnp.float32),
    )(xx)
    return jnp.transpose(out, (2, 0, 1))


def kernel(indices, weight, A, B):
    batch, hist = indices.shape
    dim = weight.shape[1]
    hp = hist // 2
    hpad = hp + (-hp) % 16
    idxh = indices.astype(jnp.int32).reshape(batch, 2, hp).transpose(1, 0, 2)
    idx3d = jnp.concatenate([idxh, idxh[:, :, : hpad - hp]], axis=2)
    fused = _fuse_table(weight, A, B)
    packed = _sc_gather(fused, idx3d)
    return _tc_transpose(packed, batch, hist, dim)
